# baseline (speedup 1.0000x reference)
import functools

import jax
import jax.numpy as jnp
from jax import lax
from jax.experimental import pallas as pl
from jax.experimental.pallas import tpu as pltpu
from jax.experimental.pallas import tpu_sc as plsc

B = 2
NPTS = 16384
K = 2048
RADII = (0.4, 0.8)
NSAMPLE = (16, 32)
_SIDE = 128
_NCHUNK = 128
_ROWS = 128
_SP = 176


def _fps_body(x_ref, y_ref, z_ref, kp_ref):
    X = x_ref[0]
    Y = y_ref[0]
    Z = z_ref[0]
    rows = lax.broadcasted_iota(jnp.int32, (_SIDE, _SIDE), 0)
    cols = lax.broadcasted_iota(jnp.int32, (_SIDE, _SIDE), 1)
    flat = rows * _SIDE + cols
    lane = lax.broadcasted_iota(jnp.int32, (1, _SIDE), 1)

    def write_kp(i, xs, ys, zs):
        row = jnp.where(lane == 0, xs,
                        jnp.where(lane == 1, ys,
                                  jnp.where(lane == 2, zs, 0.0)))
        kp_ref[0, pl.ds(i, 1), :] = row

    xs0 = X[0, 0]
    ys0 = Y[0, 0]
    zs0 = Z[0, 0]
    write_kp(0, xs0, ys0, zs0)

    def step(i, carry):
        dists, xs, ys, zs = carry
        dx = X - xs
        dy = Y - ys
        dz = Z - zs
        d = (dx * dx + dz * dz) + dy * dy
        dists = jnp.minimum(dists, d)
        m = jnp.max(dists)
        nxt = jnp.min(jnp.where(dists == m, flat, jnp.int32(1 << 30)))
        oh = flat == nxt
        nx = jnp.sum(jnp.where(oh, X, 0.0))
        ny = jnp.sum(jnp.where(oh, Y, 0.0))
        nz = jnp.sum(jnp.where(oh, Z, 0.0))
        write_kp(i, nx, ny, nz)
        return (dists, nx, ny, nz)

    dists0 = jnp.full((_SIDE, _SIDE), 1e10, dtype=jnp.float32)
    lax.fori_loop(1, K, step, (dists0, xs0, ys0, zs0))


def _fps(xyz):
    Xs = xyz[..., 0].reshape(B, _SIDE, _SIDE)
    Ys = xyz[..., 1].reshape(B, _SIDE, _SIDE)
    Zs = xyz[..., 2].reshape(B, _SIDE, _SIDE)
    return pl.pallas_call(
        _fps_body,
        grid=(B,),
        in_specs=[pl.BlockSpec((1, _SIDE, _SIDE), lambda b: (b, 0, 0))] * 3,
        out_specs=pl.BlockSpec((1, K, _SIDE), lambda b: (b, 0, 0)),
        out_shape=jax.ShapeDtypeStruct((B, K, _SIDE), jnp.float32),
    )(Xs, Ys, Zs)


def _prep_body(kp_ref, ptsT_ref, wc_ref, o0_ref, o1_ref, c0_ref, c1_ref, *,
               kb):
    kp = kp_ref[0]
    ptsT = ptsT_ref[0]
    kx = kp[:, 0]
    ky = kp[:, 1]
    kz = kp[:, 2]
    nk2 = (kx * kx + kz * kz) + ky * ky
    px = ptsT[0, :]
    py = ptsT[1, :]
    pz = ptsT[2, :]
    n2 = (px * px + pz * pz) + py * py
    dot = jnp.dot(kp, ptsT[0:3, :])
    d2 = (nk2[:, None] + n2[None, :]) - 2.0 * dot
    idx = lax.broadcasted_iota(jnp.int32, (kb, NPTS), 1)
    wc = wc_ref[...]
    for r, o_ref, c_ref in ((RADII[0], o0_ref, c0_ref),
                            (RADII[1], o1_ref, c1_ref)):
        mask = d2 < r * r
        order = jnp.where(mask, idx, jnp.int32(NPTS))
        for ch in range(_NCHUNK):
            o_ref[0, :, ch, :] = order[:, ch * 128:(ch + 1) * 128]
        cnt = jnp.dot(jnp.where(mask, 1.0, 0.0), wc)
        c_ref[0] = cnt.astype(jnp.int32)


def _prep(kp3, ptsT4, kb=64):
    r_iota = lax.broadcasted_iota(jnp.int32, (NPTS, _NCHUNK), 0)
    c_iota = lax.broadcasted_iota(jnp.int32, (NPTS, _NCHUNK), 1)
    wc = jnp.where(r_iota // _NCHUNK == c_iota, 1.0, 0.0)
    body = functools.partial(_prep_body, kb=kb)
    return pl.pallas_call(
        body,
        grid=(B, K // kb),
        in_specs=[
            pl.BlockSpec((1, kb, 3), lambda b, i: (b, i, 0)),
            pl.BlockSpec((1, 4, NPTS), lambda b, i: (b, 0, 0)),
            pl.BlockSpec((NPTS, _NCHUNK), lambda b, i: (0, 0)),
        ],
        out_specs=[
            pl.BlockSpec((1, kb, _NCHUNK, 128), lambda b, i: (b, i, 0, 0)),
            pl.BlockSpec((1, kb, _NCHUNK, 128), lambda b, i: (b, i, 0, 0)),
            pl.BlockSpec((1, kb, _NCHUNK), lambda b, i: (b, i, 0)),
            pl.BlockSpec((1, kb, _NCHUNK), lambda b, i: (b, i, 0)),
        ],
        out_shape=[
            jax.ShapeDtypeStruct((B, K, _NCHUNK, 128), jnp.int32),
            jax.ShapeDtypeStruct((B, K, _NCHUNK, 128), jnp.int32),
            jax.ShapeDtypeStruct((B, K, _NCHUNK), jnp.int32),
            jax.ShapeDtypeStruct((B, K, _NCHUNK), jnp.int32),
        ],
    )(kp3, ptsT4, wc)


def _extract_sc(order, counts, kp4, pts4, nsample):
    countsf = counts.reshape(B, K * _NCHUNK)
    kpf = kp4.reshape(B, K * 4)
    ptsf = pts4.reshape(B, NPTS * 4)
    mesh = plsc.VectorSubcoreMesh(core_axis_name="c", subcore_axis_name="s")
    nspad = nsample * 4

    def body(order_hbm, counts_hbm, kp_hbm, pts_hbm, g_hbm,
             pts_v, counts_v, kp_v, g_v, chunkdata, chunkbuf, idxbuf):
        c = lax.axis_index("c")
        s = lax.axis_index("s")
        base = s * _ROWS
        pltpu.sync_copy(pts_hbm.at[c], pts_v)
        pltpu.sync_copy(counts_hbm.at[c, pl.ds(base * _NCHUNK,
                                               _ROWS * _NCHUNK)], counts_v)
        pltpu.sync_copy(kp_hbm.at[c, pl.ds(base * 4, _ROWS * 4)], kp_v)

        iota = lax.broadcasted_iota(jnp.int32, (16,), 0)
        nvec = jnp.full((16,), NPTS, dtype=jnp.int32)
        neg = jnp.full((16,), -(1 << 30), dtype=jnp.int32)

        def row_body(r, _):
            for t in range(nsample // 16 + 1):
                idxbuf[pl.ds(t * 16, 16)] = nvec
            cpos = jnp.int32(0)
            for v8 in range(8):
                cv = counts_v[pl.ds(r * _NCHUNK + v8 * 16, 16)]
                m = cv > 0
                mi = jnp.where(m, 1, 0)
                incl = plsc.cumsum(mi)
                plsc.store_scatter(chunkbuf, [cpos + (incl - mi)],
                                   iota + v8 * 16, mask=m)
                cpos = cpos + jnp.max(incl)

            def chunk_cond(carry):
                j, pos = carry
                return jnp.logical_and(j < cpos, pos < nsample)

            def chunk_body(carry):
                j, pos = carry
                cvec = chunkbuf[pl.ds(j, 16)]
                chunk = jnp.max(jnp.where(iota == 0, cvec, neg))
                pltpu.sync_copy(order_hbm.at[c, base + r, chunk], chunkdata)
                for v in range(8):
                    ov = chunkdata[pl.ds(v * 16, 16)]
                    m = ov < NPTS
                    mi = jnp.where(m, 1, 0)
                    incl = plsc.cumsum(mi)
                    plsc.store_scatter(idxbuf, [pos + (incl - mi)], ov, mask=m)
                    pos = pos + jnp.max(incl)
                return (j + 1, pos)

            lax.while_loop(chunk_cond, chunk_body, (jnp.int32(0), jnp.int32(0)))

            kpvec = plsc.load_gather(kp_v, [r * 4 + iota % 4])
            for sg in range(nsample // 4):
                i0 = plsc.load_gather(idxbuf, [iota // 4 + sg * 4])
                msk = i0 < NPTS
                i0c = jnp.where(msk, i0, 0)
                val = plsc.load_gather(pts_v, [i0c * 4 + iota % 4])
                g = jnp.where(msk, val - kpvec, 0.0)
                g_v[pl.ds(r * nspad + sg * 16, 16)] = g
            return 0

        lax.fori_loop(0, _ROWS, row_body, 0)
        pltpu.sync_copy(g_v, g_hbm.at[c, pl.ds(base * nspad, _ROWS * nspad)])

    run = pl.kernel(
        body,
        out_type=jax.ShapeDtypeStruct((B, K * nspad), jnp.float32),
        mesh=mesh,
        compiler_params=pltpu.CompilerParams(needs_layout_passes=False),
        scratch_types=[
            pltpu.VMEM((NPTS * 4,), jnp.float32),
            pltpu.VMEM((_ROWS * _NCHUNK,), jnp.int32),
            pltpu.VMEM((_ROWS * 4,), jnp.float32),
            pltpu.VMEM((_ROWS * nspad,), jnp.float32),
            pltpu.VMEM((_NCHUNK,), jnp.int32),
            pltpu.VMEM((160,), jnp.int32),
            pltpu.VMEM((_SP,), jnp.int32),
        ],
    )
    return run(order, countsf, kpf, ptsf).reshape(B, K, nspad)


def _mlp_body(g_ref, wa_ref, wb_ref, out_ref, *, nsample, kb):
    g_all = g_ref[0]
    wa = wa_ref[...]
    wb = wb_ref[...]
    pooled = jnp.zeros((kb, 16), dtype=jnp.float32)
    for n in range(nsample):
        g = g_all[:, n * 4:(n + 1) * 4]
        h1 = jnp.maximum(jnp.dot(g, wa), 0.0)
        h2 = jnp.maximum(jnp.dot(h1, wb), 0.0)
        pooled = jnp.maximum(pooled, h2)
    out_ref[0] = pooled


def _mlp(g, Wa, Wb, nsample, kb=512):
    body = functools.partial(_mlp_body, nsample=nsample, kb=kb)
    return pl.pallas_call(
        body,
        grid=(B, K // kb),
        in_specs=[
            pl.BlockSpec((1, kb, nsample * 4), lambda b, i: (b, i, 0)),
            pl.BlockSpec((4, 16), lambda b, i: (0, 0)),
            pl.BlockSpec((16, 16), lambda b, i: (0, 0)),
        ],
        out_specs=pl.BlockSpec((1, kb, 16), lambda b, i: (b, i, 0)),
        out_shape=jax.ShapeDtypeStruct((B, K, 16), jnp.float32),
    )(g, Wa, Wb)


def kernel(points, W0a, W0b, W1a, W1b):
    pts = points.reshape(B, NPTS, 5)
    xyz = pts[:, :, 1:4]
    kp_pad = _fps(xyz)
    kp3 = kp_pad[:, :, :3]
    ptsT4 = pts[:, :, 1:5].transpose(0, 2, 1)
    o0, o1, c0, c1 = _prep(kp3, ptsT4)
    kp4 = kp_pad[:, :, :4] * jnp.array([1.0, 1.0, 1.0, 0.0])
    pts4 = pts[:, :, 1:5]
    g0 = _extract_sc(o0, c0, kp4, pts4, NSAMPLE[0])
    g1 = _extract_sc(o1, c1, kp4, pts4, NSAMPLE[1])
    f0 = _mlp(g0, W0a, W0b, NSAMPLE[0])
    f1 = _mlp(g1, W1a, W1b, NSAMPLE[1])
    point_features = jnp.concatenate([f0, f1], axis=2).reshape(B * K, 32)
    bcol = jnp.repeat(jnp.arange(B, dtype=jnp.float32), K)[:, None]
    point_coords = jnp.concatenate([bcol, kp3.reshape(B * K, 3)], axis=1)
    return point_features, point_coords


# speedup vs baseline: 1.0722x; 1.0722x over previous
import functools

import jax
import jax.numpy as jnp
from jax import lax
from jax.experimental import pallas as pl
from jax.experimental.pallas import tpu as pltpu
from jax.experimental.pallas import tpu_sc as plsc

B = 2
NPTS = 16384
K = 2048
RADII = (0.4, 0.8)
NSAMPLE = (16, 32)
_SIDE = 128
_NCHUNK = 128
_ROWS = 128
_SP = 176


def _fps_body(x_ref, y_ref, z_ref, kp_ref):
    rows = lax.broadcasted_iota(jnp.int32, (_SIDE, _SIDE), 0)
    cols = lax.broadcasted_iota(jnp.int32, (_SIDE, _SIDE), 1)
    flat = rows * _SIDE + cols
    lane = lax.broadcasted_iota(jnp.int32, (1, _SIDE), 1)
    XYZ = [(x_ref[b], y_ref[b], z_ref[b]) for b in range(B)]

    def write_kp(b, i, xs, ys, zs):
        row = jnp.where(lane == 0, xs,
                        jnp.where(lane == 1, ys,
                                  jnp.where(lane == 2, zs, 0.0)))
        kp_ref[b, pl.ds(i, 1), :] = row

    carry0 = []
    for b in range(B):
        X, Y, Z = XYZ[b]
        xs0 = X[0, 0]
        ys0 = Y[0, 0]
        zs0 = Z[0, 0]
        write_kp(b, 0, xs0, ys0, zs0)
        dists0 = jnp.full((_SIDE, _SIDE), 1e10, dtype=jnp.float32)
        carry0.extend([dists0, xs0, ys0, zs0])

    def step(i, carry):
        out = []
        for b in range(B):
            dists, xs, ys, zs = carry[4 * b:4 * b + 4]
            X, Y, Z = XYZ[b]
            dx = X - xs
            dy = Y - ys
            dz = Z - zs
            d = (dx * dx + dz * dz) + dy * dy
            dists = jnp.minimum(dists, d)
            m = jnp.max(dists)
            nxt = jnp.min(jnp.where(dists == m, flat, jnp.int32(1 << 30)))
            oh = flat == nxt
            nx = jnp.sum(jnp.where(oh, X, 0.0))
            ny = jnp.sum(jnp.where(oh, Y, 0.0))
            nz = jnp.sum(jnp.where(oh, Z, 0.0))
            write_kp(b, i, nx, ny, nz)
            out.extend([dists, nx, ny, nz])
        return tuple(out)

    lax.fori_loop(1, K, step, tuple(carry0))


def _fps(xyz):
    Xs = xyz[..., 0].reshape(B, _SIDE, _SIDE)
    Ys = xyz[..., 1].reshape(B, _SIDE, _SIDE)
    Zs = xyz[..., 2].reshape(B, _SIDE, _SIDE)
    return pl.pallas_call(
        _fps_body,
        out_shape=jax.ShapeDtypeStruct((B, K, _SIDE), jnp.float32),
    )(Xs, Ys, Zs)


def _prep_body(kp_ref, ptsT_ref, wc_ref, o0_ref, o1_ref, c0_ref, c1_ref, *,
               kb):
    kp = kp_ref[0]
    ptsT = ptsT_ref[0]
    kx = kp[:, 0]
    ky = kp[:, 1]
    kz = kp[:, 2]
    nk2 = (kx * kx + kz * kz) + ky * ky
    px = ptsT[0, :]
    py = ptsT[1, :]
    pz = ptsT[2, :]
    n2 = (px * px + pz * pz) + py * py
    dot = jnp.dot(kp, ptsT[0:3, :])
    d2 = (nk2[:, None] + n2[None, :]) - 2.0 * dot
    idx = lax.broadcasted_iota(jnp.int32, (kb, NPTS), 1)
    wc = wc_ref[...]
    for r, o_ref, c_ref in ((RADII[0], o0_ref, c0_ref),
                            (RADII[1], o1_ref, c1_ref)):
        mask = d2 < r * r
        order = jnp.where(mask, idx, jnp.int32(NPTS))
        for ch in range(_NCHUNK):
            o_ref[0, :, ch, :] = order[:, ch * 128:(ch + 1) * 128]
        cnt = jnp.dot(jnp.where(mask, 1.0, 0.0), wc)
        c_ref[0] = cnt.astype(jnp.int32)


def _prep(kp3, ptsT4, kb=64):
    r_iota = lax.broadcasted_iota(jnp.int32, (NPTS, _NCHUNK), 0)
    c_iota = lax.broadcasted_iota(jnp.int32, (NPTS, _NCHUNK), 1)
    wc = jnp.where(r_iota // _NCHUNK == c_iota, 1.0, 0.0)
    body = functools.partial(_prep_body, kb=kb)
    return pl.pallas_call(
        body,
        grid=(B, K // kb),
        in_specs=[
            pl.BlockSpec((1, kb, 3), lambda b, i: (b, i, 0)),
            pl.BlockSpec((1, 4, NPTS), lambda b, i: (b, 0, 0)),
            pl.BlockSpec((NPTS, _NCHUNK), lambda b, i: (0, 0)),
        ],
        out_specs=[
            pl.BlockSpec((1, kb, _NCHUNK, 128), lambda b, i: (b, i, 0, 0)),
            pl.BlockSpec((1, kb, _NCHUNK, 128), lambda b, i: (b, i, 0, 0)),
            pl.BlockSpec((1, kb, _NCHUNK), lambda b, i: (b, i, 0)),
            pl.BlockSpec((1, kb, _NCHUNK), lambda b, i: (b, i, 0)),
        ],
        out_shape=[
            jax.ShapeDtypeStruct((B, K, _NCHUNK, 128), jnp.int32),
            jax.ShapeDtypeStruct((B, K, _NCHUNK, 128), jnp.int32),
            jax.ShapeDtypeStruct((B, K, _NCHUNK), jnp.int32),
            jax.ShapeDtypeStruct((B, K, _NCHUNK), jnp.int32),
        ],
    )(kp3, ptsT4, wc)


def _extract_sc(order, counts, kp4, pts4, nsample):
    countsf = counts.reshape(B, K * _NCHUNK)
    kpf = kp4.reshape(B, K * 4)
    ptsf = pts4.reshape(B, NPTS * 4)
    mesh = plsc.VectorSubcoreMesh(core_axis_name="c", subcore_axis_name="s")
    nspad = nsample * 4

    def body(order_hbm, counts_hbm, kp_hbm, pts_hbm, g_hbm,
             pts_v, counts_v, kp_v, g_v, chunkdata, chunkbuf, idxbuf):
        c = lax.axis_index("c")
        s = lax.axis_index("s")
        base = s * _ROWS
        pltpu.sync_copy(pts_hbm.at[c], pts_v)
        pltpu.sync_copy(counts_hbm.at[c, pl.ds(base * _NCHUNK,
                                               _ROWS * _NCHUNK)], counts_v)
        pltpu.sync_copy(kp_hbm.at[c, pl.ds(base * 4, _ROWS * 4)], kp_v)

        iota = lax.broadcasted_iota(jnp.int32, (16,), 0)
        nvec = jnp.full((16,), NPTS, dtype=jnp.int32)
        neg = jnp.full((16,), -(1 << 30), dtype=jnp.int32)

        def row_body(r, _):
            for t in range(nsample // 16 + 1):
                idxbuf[pl.ds(t * 16, 16)] = nvec
            cpos = jnp.int32(0)
            for v8 in range(8):
                cv = counts_v[pl.ds(r * _NCHUNK + v8 * 16, 16)]
                m = cv > 0
                mi = jnp.where(m, 1, 0)
                incl = plsc.cumsum(mi)
                plsc.store_scatter(chunkbuf, [cpos + (incl - mi)],
                                   iota + v8 * 16, mask=m)
                cpos = cpos + jnp.max(incl)

            def chunk_cond(carry):
                j, pos = carry
                return jnp.logical_and(j < cpos, pos < nsample)

            def chunk_body(carry):
                j, pos = carry
                cvec = chunkbuf[pl.ds(j, 16)]
                chunk = jnp.max(jnp.where(iota == 0, cvec, neg))
                pltpu.sync_copy(order_hbm.at[c, base + r, chunk], chunkdata)
                for v in range(8):
                    ov = chunkdata[pl.ds(v * 16, 16)]
                    m = ov < NPTS
                    mi = jnp.where(m, 1, 0)
                    incl = plsc.cumsum(mi)
                    plsc.store_scatter(idxbuf, [pos + (incl - mi)], ov, mask=m)
                    pos = pos + jnp.max(incl)
                return (j + 1, pos)

            lax.while_loop(chunk_cond, chunk_body, (jnp.int32(0), jnp.int32(0)))

            kpvec = plsc.load_gather(kp_v, [r * 4 + iota % 4])
            for sg in range(nsample // 4):
                i0 = plsc.load_gather(idxbuf, [iota // 4 + sg * 4])
                msk = i0 < NPTS
                i0c = jnp.where(msk, i0, 0)
                val = plsc.load_gather(pts_v, [i0c * 4 + iota % 4])
                g = jnp.where(msk, val - kpvec, 0.0)
                g_v[pl.ds(r * nspad + sg * 16, 16)] = g
            return 0

        lax.fori_loop(0, _ROWS, row_body, 0)
        pltpu.sync_copy(g_v, g_hbm.at[c, pl.ds(base * nspad, _ROWS * nspad)])

    run = pl.kernel(
        body,
        out_type=jax.ShapeDtypeStruct((B, K * nspad), jnp.float32),
        mesh=mesh,
        compiler_params=pltpu.CompilerParams(needs_layout_passes=False),
        scratch_types=[
            pltpu.VMEM((NPTS * 4,), jnp.float32),
            pltpu.VMEM((_ROWS * _NCHUNK,), jnp.int32),
            pltpu.VMEM((_ROWS * 4,), jnp.float32),
            pltpu.VMEM((_ROWS * nspad,), jnp.float32),
            pltpu.VMEM((_NCHUNK,), jnp.int32),
            pltpu.VMEM((160,), jnp.int32),
            pltpu.VMEM((_SP,), jnp.int32),
        ],
    )
    return run(order, countsf, kpf, ptsf).reshape(B, K, nspad)


def _mlp_body(g_ref, wa_ref, wb_ref, out_ref, *, nsample, kb):
    g_all = g_ref[0]
    wa = wa_ref[...]
    wb = wb_ref[...]
    pooled = jnp.zeros((kb, 16), dtype=jnp.float32)
    for n in range(nsample):
        g = g_all[:, n * 4:(n + 1) * 4]
        h1 = jnp.maximum(jnp.dot(g, wa), 0.0)
        h2 = jnp.maximum(jnp.dot(h1, wb), 0.0)
        pooled = jnp.maximum(pooled, h2)
    out_ref[0] = pooled


def _mlp(g, Wa, Wb, nsample, kb=512):
    body = functools.partial(_mlp_body, nsample=nsample, kb=kb)
    return pl.pallas_call(
        body,
        grid=(B, K // kb),
        in_specs=[
            pl.BlockSpec((1, kb, nsample * 4), lambda b, i: (b, i, 0)),
            pl.BlockSpec((4, 16), lambda b, i: (0, 0)),
            pl.BlockSpec((16, 16), lambda b, i: (0, 0)),
        ],
        out_specs=pl.BlockSpec((1, kb, 16), lambda b, i: (b, i, 0)),
        out_shape=jax.ShapeDtypeStruct((B, K, 16), jnp.float32),
    )(g, Wa, Wb)


def kernel(points, W0a, W0b, W1a, W1b):
    pts = points.reshape(B, NPTS, 5)
    xyz = pts[:, :, 1:4]
    kp_pad = _fps(xyz)
    kp3 = kp_pad[:, :, :3]
    ptsT4 = pts[:, :, 1:5].transpose(0, 2, 1)
    o0, o1, c0, c1 = _prep(kp3, ptsT4)
    kp4 = kp_pad[:, :, :4] * jnp.array([1.0, 1.0, 1.0, 0.0])
    pts4 = pts[:, :, 1:5]
    g0 = _extract_sc(o0, c0, kp4, pts4, NSAMPLE[0])
    g1 = _extract_sc(o1, c1, kp4, pts4, NSAMPLE[1])
    f0 = _mlp(g0, W0a, W0b, NSAMPLE[0])
    f1 = _mlp(g1, W1a, W1b, NSAMPLE[1])
    point_features = jnp.concatenate([f0, f1], axis=2).reshape(B * K, 32)
    bcol = jnp.repeat(jnp.arange(B, dtype=jnp.float32), K)[:, None]
    point_coords = jnp.concatenate([bcol, kp3.reshape(B * K, 3)], axis=1)
    return point_features, point_coords


# speedup vs baseline: 1.0837x; 1.0107x over previous
import functools

import jax
import jax.numpy as jnp
from jax import lax
from jax.experimental import pallas as pl
from jax.experimental.pallas import tpu as pltpu
from jax.experimental.pallas import tpu_sc as plsc

B = 2
NPTS = 16384
K = 2048
RADII = (0.4, 0.8)
NSAMPLE = (16, 32)
_SIDE = 128
_NCHUNK = 128
_ROWS = 128
_SP = 176


def _fps_body(x_ref, y_ref, z_ref, kp_ref):
    rows = lax.broadcasted_iota(jnp.int32, (_SIDE, _SIDE), 0)
    cols = lax.broadcasted_iota(jnp.int32, (_SIDE, _SIDE), 1)
    flat = rows * _SIDE + cols
    lane = lax.broadcasted_iota(jnp.int32, (1, _SIDE), 1)
    XYZ = [(x_ref[b], y_ref[b], z_ref[b]) for b in range(B)]

    def write_kp(b, i, xs, ys, zs):
        row = jnp.where(lane == 0, xs,
                        jnp.where(lane == 1, ys,
                                  jnp.where(lane == 2, zs, 0.0)))
        kp_ref[b, pl.ds(i, 1), :] = row

    carry0 = []
    for b in range(B):
        X, Y, Z = XYZ[b]
        xs0 = X[0, 0]
        ys0 = Y[0, 0]
        zs0 = Z[0, 0]
        write_kp(b, 0, xs0, ys0, zs0)
        dists0 = jnp.full((_SIDE, _SIDE), 1e10, dtype=jnp.float32)
        carry0.extend([dists0, xs0, ys0, zs0])

    def step(i, carry):
        out = []
        for b in range(B):
            dists, xs, ys, zs = carry[4 * b:4 * b + 4]
            X, Y, Z = XYZ[b]
            dx = X - xs
            dy = Y - ys
            dz = Z - zs
            d = (dx * dx + dz * dz) + dy * dy
            dists = jnp.minimum(dists, d)
            m = jnp.max(dists)
            nxt = jnp.min(jnp.where(dists == m, flat, jnp.int32(1 << 30)))
            r = nxt // _SIDE
            col = nxt % _SIDE
            lm = lane == col
            nx = jnp.sum(jnp.where(lm, x_ref[b, pl.ds(r, 1), :], 0.0))
            ny = jnp.sum(jnp.where(lm, y_ref[b, pl.ds(r, 1), :], 0.0))
            nz = jnp.sum(jnp.where(lm, z_ref[b, pl.ds(r, 1), :], 0.0))
            write_kp(b, i, nx, ny, nz)
            out.extend([dists, nx, ny, nz])
        return tuple(out)

    lax.fori_loop(1, K, step, tuple(carry0))


def _fps(xyz):
    Xs = xyz[..., 0].reshape(B, _SIDE, _SIDE)
    Ys = xyz[..., 1].reshape(B, _SIDE, _SIDE)
    Zs = xyz[..., 2].reshape(B, _SIDE, _SIDE)
    return pl.pallas_call(
        _fps_body,
        out_shape=jax.ShapeDtypeStruct((B, K, _SIDE), jnp.float32),
    )(Xs, Ys, Zs)


def _prep_body(kp_ref, ptsT_ref, wc_ref, o0_ref, o1_ref, c0_ref, c1_ref, *,
               kb):
    kp = kp_ref[0]
    ptsT = ptsT_ref[0]
    kx = kp[:, 0]
    ky = kp[:, 1]
    kz = kp[:, 2]
    nk2 = (kx * kx + kz * kz) + ky * ky
    px = ptsT[0, :]
    py = ptsT[1, :]
    pz = ptsT[2, :]
    n2 = (px * px + pz * pz) + py * py
    dot = jnp.dot(kp, ptsT[0:3, :])
    d2 = (nk2[:, None] + n2[None, :]) - 2.0 * dot
    idx = lax.broadcasted_iota(jnp.int32, (kb, NPTS), 1)
    wc = wc_ref[...]
    for r, o_ref, c_ref in ((RADII[0], o0_ref, c0_ref),
                            (RADII[1], o1_ref, c1_ref)):
        mask = d2 < r * r
        order = jnp.where(mask, idx, jnp.int32(NPTS))
        for ch in range(_NCHUNK):
            o_ref[0, :, ch, :] = order[:, ch * 128:(ch + 1) * 128]
        cnt = jnp.dot(jnp.where(mask, 1.0, 0.0), wc)
        c_ref[0] = cnt.astype(jnp.int32)


def _prep(kp3, ptsT4, kb=64):
    r_iota = lax.broadcasted_iota(jnp.int32, (NPTS, _NCHUNK), 0)
    c_iota = lax.broadcasted_iota(jnp.int32, (NPTS, _NCHUNK), 1)
    wc = jnp.where(r_iota // _NCHUNK == c_iota, 1.0, 0.0)
    body = functools.partial(_prep_body, kb=kb)
    return pl.pallas_call(
        body,
        grid=(B, K // kb),
        in_specs=[
            pl.BlockSpec((1, kb, 3), lambda b, i: (b, i, 0)),
            pl.BlockSpec((1, 4, NPTS), lambda b, i: (b, 0, 0)),
            pl.BlockSpec((NPTS, _NCHUNK), lambda b, i: (0, 0)),
        ],
        out_specs=[
            pl.BlockSpec((1, kb, _NCHUNK, 128), lambda b, i: (b, i, 0, 0)),
            pl.BlockSpec((1, kb, _NCHUNK, 128), lambda b, i: (b, i, 0, 0)),
            pl.BlockSpec((1, kb, _NCHUNK), lambda b, i: (b, i, 0)),
            pl.BlockSpec((1, kb, _NCHUNK), lambda b, i: (b, i, 0)),
        ],
        out_shape=[
            jax.ShapeDtypeStruct((B, K, _NCHUNK, 128), jnp.int32),
            jax.ShapeDtypeStruct((B, K, _NCHUNK, 128), jnp.int32),
            jax.ShapeDtypeStruct((B, K, _NCHUNK), jnp.int32),
            jax.ShapeDtypeStruct((B, K, _NCHUNK), jnp.int32),
        ],
    )(kp3, ptsT4, wc)


def _extract_sc(order, counts, kp4, pts4, nsample):
    countsf = counts.reshape(B, K * _NCHUNK)
    kpf = kp4.reshape(B, K * 4)
    ptsf = pts4.reshape(B, NPTS * 4)
    mesh = plsc.VectorSubcoreMesh(core_axis_name="c", subcore_axis_name="s")
    nspad = nsample * 4

    def body(order_hbm, counts_hbm, kp_hbm, pts_hbm, g_hbm,
             pts_v, counts_v, kp_v, g_v, chunkdata, chunkbuf, idxbuf):
        c = lax.axis_index("c")
        s = lax.axis_index("s")
        base = s * _ROWS
        pltpu.sync_copy(pts_hbm.at[c], pts_v)
        pltpu.sync_copy(counts_hbm.at[c, pl.ds(base * _NCHUNK,
                                               _ROWS * _NCHUNK)], counts_v)
        pltpu.sync_copy(kp_hbm.at[c, pl.ds(base * 4, _ROWS * 4)], kp_v)

        iota = lax.broadcasted_iota(jnp.int32, (16,), 0)
        nvec = jnp.full((16,), NPTS, dtype=jnp.int32)
        neg = jnp.full((16,), -(1 << 30), dtype=jnp.int32)

        def row_body(r, _):
            for t in range(nsample // 16 + 1):
                idxbuf[pl.ds(t * 16, 16)] = nvec
            cpos = jnp.int32(0)
            for v8 in range(8):
                cv = counts_v[pl.ds(r * _NCHUNK + v8 * 16, 16)]
                m = cv > 0
                mi = jnp.where(m, 1, 0)
                incl = plsc.cumsum(mi)
                plsc.store_scatter(chunkbuf, [cpos + (incl - mi)],
                                   iota + v8 * 16, mask=m)
                cpos = cpos + jnp.max(incl)

            def chunk_cond(carry):
                j, pos = carry
                return jnp.logical_and(j < cpos, pos < nsample)

            def chunk_body(carry):
                j, pos = carry
                cvec = chunkbuf[pl.ds(j, 16)]
                chunk = jnp.max(jnp.where(iota == 0, cvec, neg))
                pltpu.sync_copy(order_hbm.at[c, base + r, chunk], chunkdata)
                for v in range(8):
                    ov = chunkdata[pl.ds(v * 16, 16)]
                    m = ov < NPTS
                    mi = jnp.where(m, 1, 0)
                    incl = plsc.cumsum(mi)
                    plsc.store_scatter(idxbuf, [pos + (incl - mi)], ov, mask=m)
                    pos = pos + jnp.max(incl)
                return (j + 1, pos)

            lax.while_loop(chunk_cond, chunk_body, (jnp.int32(0), jnp.int32(0)))

            kpvec = plsc.load_gather(kp_v, [r * 4 + iota % 4])
            for sg in range(nsample // 4):
                i0 = plsc.load_gather(idxbuf, [iota // 4 + sg * 4])
                msk = i0 < NPTS
                i0c = jnp.where(msk, i0, 0)
                val = plsc.load_gather(pts_v, [i0c * 4 + iota % 4])
                g = jnp.where(msk, val - kpvec, 0.0)
                g_v[pl.ds(r * nspad + sg * 16, 16)] = g
            return 0

        lax.fori_loop(0, _ROWS, row_body, 0)
        pltpu.sync_copy(g_v, g_hbm.at[c, pl.ds(base * nspad, _ROWS * nspad)])

    run = pl.kernel(
        body,
        out_type=jax.ShapeDtypeStruct((B, K * nspad), jnp.float32),
        mesh=mesh,
        compiler_params=pltpu.CompilerParams(needs_layout_passes=False),
        scratch_types=[
            pltpu.VMEM((NPTS * 4,), jnp.float32),
            pltpu.VMEM((_ROWS * _NCHUNK,), jnp.int32),
            pltpu.VMEM((_ROWS * 4,), jnp.float32),
            pltpu.VMEM((_ROWS * nspad,), jnp.float32),
            pltpu.VMEM((_NCHUNK,), jnp.int32),
            pltpu.VMEM((160,), jnp.int32),
            pltpu.VMEM((_SP,), jnp.int32),
        ],
    )
    return run(order, countsf, kpf, ptsf).reshape(B, K, nspad)


def _mlp_body(g_ref, wa_ref, wb_ref, out_ref, *, nsample, kb):
    g_all = g_ref[0]
    wa = wa_ref[...]
    wb = wb_ref[...]
    pooled = jnp.zeros((kb, 16), dtype=jnp.float32)
    for n in range(nsample):
        g = g_all[:, n * 4:(n + 1) * 4]
        h1 = jnp.maximum(jnp.dot(g, wa), 0.0)
        h2 = jnp.maximum(jnp.dot(h1, wb), 0.0)
        pooled = jnp.maximum(pooled, h2)
    out_ref[0] = pooled


def _mlp(g, Wa, Wb, nsample, kb=512):
    body = functools.partial(_mlp_body, nsample=nsample, kb=kb)
    return pl.pallas_call(
        body,
        grid=(B, K // kb),
        in_specs=[
            pl.BlockSpec((1, kb, nsample * 4), lambda b, i: (b, i, 0)),
            pl.BlockSpec((4, 16), lambda b, i: (0, 0)),
            pl.BlockSpec((16, 16), lambda b, i: (0, 0)),
        ],
        out_specs=pl.BlockSpec((1, kb, 16), lambda b, i: (b, i, 0)),
        out_shape=jax.ShapeDtypeStruct((B, K, 16), jnp.float32),
    )(g, Wa, Wb)


def kernel(points, W0a, W0b, W1a, W1b):
    pts = points.reshape(B, NPTS, 5)
    xyz = pts[:, :, 1:4]
    kp_pad = _fps(xyz)
    kp3 = kp_pad[:, :, :3]
    ptsT4 = pts[:, :, 1:5].transpose(0, 2, 1)
    o0, o1, c0, c1 = _prep(kp3, ptsT4)
    kp4 = kp_pad[:, :, :4] * jnp.array([1.0, 1.0, 1.0, 0.0])
    pts4 = pts[:, :, 1:5]
    g0 = _extract_sc(o0, c0, kp4, pts4, NSAMPLE[0])
    g1 = _extract_sc(o1, c1, kp4, pts4, NSAMPLE[1])
    f0 = _mlp(g0, W0a, W0b, NSAMPLE[0])
    f1 = _mlp(g1, W1a, W1b, NSAMPLE[1])
    point_features = jnp.concatenate([f0, f1], axis=2).reshape(B * K, 32)
    bcol = jnp.repeat(jnp.arange(B, dtype=jnp.float32), K)[:, None]
    point_coords = jnp.concatenate([bcol, kp3.reshape(B * K, 3)], axis=1)
    return point_features, point_coords


# speedup vs baseline: 1.1354x; 1.0478x over previous
import functools

import jax
import jax.numpy as jnp
from jax import lax
from jax.experimental import pallas as pl
from jax.experimental.pallas import tpu as pltpu
from jax.experimental.pallas import tpu_sc as plsc

B = 2
NPTS = 16384
K = 2048
RADII = (0.4, 0.8)
NSAMPLE = (16, 32)
_SIDE = 128
_NCHUNK = 128
_ROWS = 128
_SP = 304


def _fps_body(x_ref, y_ref, z_ref, kp_ref):
    rows = lax.broadcasted_iota(jnp.int32, (_SIDE, _SIDE), 0)
    cols = lax.broadcasted_iota(jnp.int32, (_SIDE, _SIDE), 1)
    flat = rows * _SIDE + cols
    lane = lax.broadcasted_iota(jnp.int32, (1, _SIDE), 1)
    XYZ = [(x_ref[b], y_ref[b], z_ref[b]) for b in range(B)]

    def write_kp(b, i, xs, ys, zs):
        row = jnp.where(lane == 0, xs,
                        jnp.where(lane == 1, ys,
                                  jnp.where(lane == 2, zs, 0.0)))
        kp_ref[b, pl.ds(i, 1), :] = row

    carry0 = []
    for b in range(B):
        X, Y, Z = XYZ[b]
        xs0 = X[0, 0]
        ys0 = Y[0, 0]
        zs0 = Z[0, 0]
        write_kp(b, 0, xs0, ys0, zs0)
        dists0 = jnp.full((_SIDE, _SIDE), 1e10, dtype=jnp.float32)
        carry0.extend([dists0, xs0, ys0, zs0])

    def step(i, carry):
        out = []
        for b in range(B):
            dists, xs, ys, zs = carry[4 * b:4 * b + 4]
            X, Y, Z = XYZ[b]
            dx = X - xs
            dy = Y - ys
            dz = Z - zs
            d = (dx * dx + dz * dz) + dy * dy
            dists = jnp.minimum(dists, d)
            m = jnp.max(dists)
            nxt = jnp.min(jnp.where(dists == m, flat, jnp.int32(1 << 30)))
            r = nxt // _SIDE
            col = nxt % _SIDE
            lm = lane == col
            nx = jnp.sum(jnp.where(lm, x_ref[b, pl.ds(r, 1), :], 0.0))
            ny = jnp.sum(jnp.where(lm, y_ref[b, pl.ds(r, 1), :], 0.0))
            nz = jnp.sum(jnp.where(lm, z_ref[b, pl.ds(r, 1), :], 0.0))
            write_kp(b, i, nx, ny, nz)
            out.extend([dists, nx, ny, nz])
        return tuple(out)

    lax.fori_loop(1, K, step, tuple(carry0))


def _fps(xyz):
    Xs = xyz[..., 0].reshape(B, _SIDE, _SIDE)
    Ys = xyz[..., 1].reshape(B, _SIDE, _SIDE)
    Zs = xyz[..., 2].reshape(B, _SIDE, _SIDE)
    return pl.pallas_call(
        _fps_body,
        out_shape=jax.ShapeDtypeStruct((B, K, _SIDE), jnp.float32),
    )(Xs, Ys, Zs)


def _prep_body(kp_ref, ptsT_ref, wc_ref, o0_ref, o1_ref, c0_ref, c1_ref, *,
               kb):
    kp = kp_ref[0]
    ptsT = ptsT_ref[0]
    kx = kp[:, 0]
    ky = kp[:, 1]
    kz = kp[:, 2]
    nk2 = (kx * kx + kz * kz) + ky * ky
    px = ptsT[0, :]
    py = ptsT[1, :]
    pz = ptsT[2, :]
    n2 = (px * px + pz * pz) + py * py
    dot = jnp.dot(kp, ptsT[0:3, :])
    d2 = (nk2[:, None] + n2[None, :]) - 2.0 * dot
    idx = lax.broadcasted_iota(jnp.int32, (kb, NPTS), 1)
    wc = wc_ref[...]
    for r, o_ref, c_ref in ((RADII[0], o0_ref, c0_ref),
                            (RADII[1], o1_ref, c1_ref)):
        mask = d2 < r * r
        order = jnp.where(mask, idx, jnp.int32(NPTS))
        for ch in range(_NCHUNK):
            o_ref[0, :, ch, :] = order[:, ch * 128:(ch + 1) * 128]
        cnt = jnp.dot(jnp.where(mask, 1.0, 0.0), wc)
        c_ref[0] = cnt.astype(jnp.int32)


def _prep(kp3, ptsT4, kb=64):
    r_iota = lax.broadcasted_iota(jnp.int32, (NPTS, _NCHUNK), 0)
    c_iota = lax.broadcasted_iota(jnp.int32, (NPTS, _NCHUNK), 1)
    wc = jnp.where(r_iota // _NCHUNK == c_iota, 1.0, 0.0)
    body = functools.partial(_prep_body, kb=kb)
    return pl.pallas_call(
        body,
        grid=(B, K // kb),
        in_specs=[
            pl.BlockSpec((1, kb, 3), lambda b, i: (b, i, 0)),
            pl.BlockSpec((1, 4, NPTS), lambda b, i: (b, 0, 0)),
            pl.BlockSpec((NPTS, _NCHUNK), lambda b, i: (0, 0)),
        ],
        out_specs=[
            pl.BlockSpec((1, kb, _NCHUNK, 128), lambda b, i: (b, i, 0, 0)),
            pl.BlockSpec((1, kb, _NCHUNK, 128), lambda b, i: (b, i, 0, 0)),
            pl.BlockSpec((1, kb, _NCHUNK), lambda b, i: (b, i, 0)),
            pl.BlockSpec((1, kb, _NCHUNK), lambda b, i: (b, i, 0)),
        ],
        out_shape=[
            jax.ShapeDtypeStruct((B, K, _NCHUNK, 128), jnp.int32),
            jax.ShapeDtypeStruct((B, K, _NCHUNK, 128), jnp.int32),
            jax.ShapeDtypeStruct((B, K, _NCHUNK), jnp.int32),
            jax.ShapeDtypeStruct((B, K, _NCHUNK), jnp.int32),
        ],
    )(kp3, ptsT4, wc)


def _extract_sc(order, counts, kp4, pts4, nsample):
    countsf = counts.reshape(B, K * _NCHUNK)
    kpf = kp4.reshape(B, K * 4)
    ptsf = pts4.reshape(B, NPTS * 4)
    mesh = plsc.VectorSubcoreMesh(core_axis_name="c", subcore_axis_name="s")
    nspad = nsample * 4

    def body(order_hbm, counts_hbm, kp_hbm, pts_hbm, g_hbm,
             pts_v, counts_v, kp_v, g_v, chunkdata,
             stage_a, stage_b, cbuf_a, cbuf_b, idxbuf, sem_a, sem_b):
        c = lax.axis_index("c")
        s = lax.axis_index("s")
        base = s * _ROWS
        pltpu.sync_copy(pts_hbm.at[c], pts_v)
        pltpu.sync_copy(counts_hbm.at[c, pl.ds(base * _NCHUNK,
                                               _ROWS * _NCHUNK)], counts_v)
        pltpu.sync_copy(kp_hbm.at[c, pl.ds(base * 4, _ROWS * 4)], kp_v)

        iota = lax.broadcasted_iota(jnp.int32, (16,), 0)
        nvec = jnp.full((16,), NPTS, dtype=jnp.int32)
        neg = jnp.full((16,), -(1 << 30), dtype=jnp.int32)

        def cextract(cbuf, j):
            cvec = cbuf[pl.ds(j, 16)]
            return jnp.max(jnp.where(iota == 0, cvec, neg))

        def build_and_fire(r, cbuf, stage, sem):
            cpos = jnp.int32(0)
            for v8 in range(8):
                cv = counts_v[pl.ds(r * _NCHUNK + v8 * 16, 16)]
                m = cv > 0
                mi = jnp.where(m, 1, 0)
                incl = plsc.cumsum(mi)
                plsc.store_scatter(cbuf, [cpos + (incl - mi)],
                                   iota + v8 * 16, mask=m)
                cpos = cpos + jnp.max(incl)
            ch0 = jnp.clip(cextract(cbuf, 0), 0, _NCHUNK - 1)
            ch1 = jnp.clip(cextract(cbuf, 1), 0, _NCHUNK - 1)
            pltpu.async_copy(order_hbm.at[c, base + r, ch0],
                             stage.at[pl.ds(0, 128)], sem)
            pltpu.async_copy(order_hbm.at[c, base + r, ch1],
                             stage.at[pl.ds(128, 128)], sem)
            return cpos

        def process(r, cbuf, stage, sem, nch):
            pltpu.make_async_copy(order_hbm.at[c, 0, 0],
                                  stage.at[pl.ds(0, 128)], sem).wait()
            pltpu.make_async_copy(order_hbm.at[c, 0, 0],
                                  stage.at[pl.ds(128, 128)], sem).wait()
            for t in range(nsample // 16 + 1):
                idxbuf[pl.ds(t * 16, 16)] = nvec
            pos = jnp.int32(0)
            for t in range(2):
                live = t < nch
                for v in range(8):
                    ov = stage[pl.ds(t * 128 + v * 16, 16)]
                    m = jnp.logical_and(ov < NPTS, live)
                    mi = jnp.where(m, 1, 0)
                    incl = plsc.cumsum(mi)
                    plsc.store_scatter(idxbuf, [pos + (incl - mi)], ov,
                                       mask=m)
                    pos = pos + jnp.max(incl)

            def chunk_cond(carry):
                j, pos = carry
                return jnp.logical_and(j < nch, pos < nsample)

            def chunk_body(carry):
                j, pos = carry
                chunk = cextract(cbuf, j)
                pltpu.sync_copy(order_hbm.at[c, base + r, chunk], chunkdata)
                for v in range(8):
                    ov = chunkdata[pl.ds(v * 16, 16)]
                    m = ov < NPTS
                    mi = jnp.where(m, 1, 0)
                    incl = plsc.cumsum(mi)
                    plsc.store_scatter(idxbuf, [pos + (incl - mi)], ov,
                                       mask=m)
                    pos = pos + jnp.max(incl)
                return (j + 1, pos)

            lax.while_loop(chunk_cond, chunk_body, (jnp.int32(2), pos))

            kpvec = plsc.load_gather(kp_v, [r * 4 + iota % 4])
            for sg in range(nsample // 4):
                i0 = plsc.load_gather(idxbuf, [iota // 4 + sg * 4])
                msk = i0 < NPTS
                i0c = jnp.where(msk, i0, 0)
                val = plsc.load_gather(pts_v, [i0c * 4 + iota % 4])
                g = jnp.where(msk, val - kpvec, 0.0)
                g_v[pl.ds(r * nspad + sg * 16, 16)] = g

        nch0 = build_and_fire(0, cbuf_a, stage_a, sem_a)

        def pair_body(gidx, nch_a):
            r0 = gidx * 2
            nch_b = build_and_fire(r0 + 1, cbuf_b, stage_b, sem_b)
            process(r0, cbuf_a, stage_a, sem_a, nch_a)
            r2 = jnp.minimum(r0 + 2, _ROWS - 1)
            nch_a2 = build_and_fire(r2, cbuf_a, stage_a, sem_a)
            process(r0 + 1, cbuf_b, stage_b, sem_b, nch_b)
            return nch_a2

        lax.fori_loop(0, _ROWS // 2, pair_body, nch0)
        pltpu.make_async_copy(order_hbm.at[c, 0, 0],
                              stage_a.at[pl.ds(0, 128)], sem_a).wait()
        pltpu.make_async_copy(order_hbm.at[c, 0, 0],
                              stage_a.at[pl.ds(128, 128)], sem_a).wait()
        pltpu.sync_copy(g_v, g_hbm.at[c, pl.ds(base * nspad, _ROWS * nspad)])

    run = pl.kernel(
        body,
        out_type=jax.ShapeDtypeStruct((B, K * nspad), jnp.float32),
        mesh=mesh,
        compiler_params=pltpu.CompilerParams(needs_layout_passes=False),
        scratch_types=[
            pltpu.VMEM((NPTS * 4,), jnp.float32),
            pltpu.VMEM((_ROWS * _NCHUNK,), jnp.int32),
            pltpu.VMEM((_ROWS * 4,), jnp.float32),
            pltpu.VMEM((_ROWS * nspad,), jnp.float32),
            pltpu.VMEM((_NCHUNK,), jnp.int32),
            pltpu.VMEM((256,), jnp.int32),
            pltpu.VMEM((256,), jnp.int32),
            pltpu.VMEM((160,), jnp.int32),
            pltpu.VMEM((160,), jnp.int32),
            pltpu.VMEM((_SP,), jnp.int32),
            pltpu.SemaphoreType.DMA,
            pltpu.SemaphoreType.DMA,
        ],
    )
    return run(order, countsf, kpf, ptsf).reshape(B, K, nspad)


def _mlp_body(g_ref, wa_ref, wb_ref, out_ref, *, nsample, kb):
    g_all = g_ref[0]
    wa = wa_ref[...]
    wb = wb_ref[...]
    pooled = jnp.zeros((kb, 16), dtype=jnp.float32)
    for n in range(nsample):
        g = g_all[:, n * 4:(n + 1) * 4]
        h1 = jnp.maximum(jnp.dot(g, wa), 0.0)
        h2 = jnp.maximum(jnp.dot(h1, wb), 0.0)
        pooled = jnp.maximum(pooled, h2)
    out_ref[0] = pooled


def _mlp(g, Wa, Wb, nsample, kb=512):
    body = functools.partial(_mlp_body, nsample=nsample, kb=kb)
    return pl.pallas_call(
        body,
        grid=(B, K // kb),
        in_specs=[
            pl.BlockSpec((1, kb, nsample * 4), lambda b, i: (b, i, 0)),
            pl.BlockSpec((4, 16), lambda b, i: (0, 0)),
            pl.BlockSpec((16, 16), lambda b, i: (0, 0)),
        ],
        out_specs=pl.BlockSpec((1, kb, 16), lambda b, i: (b, i, 0)),
        out_shape=jax.ShapeDtypeStruct((B, K, 16), jnp.float32),
    )(g, Wa, Wb)


def kernel(points, W0a, W0b, W1a, W1b):
    pts = points.reshape(B, NPTS, 5)
    xyz = pts[:, :, 1:4]
    kp_pad = _fps(xyz)
    kp3 = kp_pad[:, :, :3]
    ptsT4 = pts[:, :, 1:5].transpose(0, 2, 1)
    o0, o1, c0, c1 = _prep(kp3, ptsT4)
    kp4 = kp_pad[:, :, :4] * jnp.array([1.0, 1.0, 1.0, 0.0])
    pts4 = pts[:, :, 1:5]
    g0 = _extract_sc(o0, c0, kp4, pts4, NSAMPLE[0])
    g1 = _extract_sc(o1, c1, kp4, pts4, NSAMPLE[1])
    f0 = _mlp(g0, W0a, W0b, NSAMPLE[0])
    f1 = _mlp(g1, W1a, W1b, NSAMPLE[1])
    point_features = jnp.concatenate([f0, f1], axis=2).reshape(B * K, 32)
    bcol = jnp.repeat(jnp.arange(B, dtype=jnp.float32), K)[:, None]
    point_coords = jnp.concatenate([bcol, kp3.reshape(B * K, 3)], axis=1)
    return point_features, point_coords


# speedup vs baseline: 1.2177x; 1.0725x over previous
import functools

import jax
import jax.numpy as jnp
from jax import lax
from jax.experimental import pallas as pl
from jax.experimental.pallas import tpu as pltpu
from jax.experimental.pallas import tpu_sc as plsc

B = 2
NPTS = 16384
K = 2048
RADII = (0.4, 0.8)
NSAMPLE = (16, 32)
_SIDE = 128
_NCHUNK = 128
_ROWS = 128
_SP = 304


def _fps_body(x_ref, y_ref, z_ref, kp_ref):
    rows = lax.broadcasted_iota(jnp.int32, (_SIDE, _SIDE), 0)
    cols = lax.broadcasted_iota(jnp.int32, (_SIDE, _SIDE), 1)
    flat = rows * _SIDE + cols
    lane = lax.broadcasted_iota(jnp.int32, (1, _SIDE), 1)
    XYZ = [(x_ref[b], y_ref[b], z_ref[b]) for b in range(B)]

    def write_kp(b, i, xs, ys, zs):
        row = jnp.where(lane == 0, xs,
                        jnp.where(lane == 1, ys,
                                  jnp.where(lane == 2, zs, 0.0)))
        kp_ref[b, pl.ds(i, 1), :] = row

    carry0 = []
    for b in range(B):
        X, Y, Z = XYZ[b]
        xs0 = X[0, 0]
        ys0 = Y[0, 0]
        zs0 = Z[0, 0]
        write_kp(b, 0, xs0, ys0, zs0)
        dists0 = jnp.full((_SIDE, _SIDE), 1e10, dtype=jnp.float32)
        carry0.extend([dists0, xs0, ys0, zs0])

    def step(i, carry):
        out = []
        for b in range(B):
            dists, xs, ys, zs = carry[4 * b:4 * b + 4]
            X, Y, Z = XYZ[b]
            dx = X - xs
            dy = Y - ys
            dz = Z - zs
            d = (dx * dx + dz * dz) + dy * dy
            dists = jnp.minimum(dists, d)
            m = jnp.max(dists)
            nxt = jnp.min(jnp.where(dists == m, flat, jnp.int32(1 << 30)))
            r = nxt // _SIDE
            col = nxt % _SIDE
            lm = lane == col
            nx = jnp.sum(jnp.where(lm, x_ref[b, pl.ds(r, 1), :], 0.0))
            ny = jnp.sum(jnp.where(lm, y_ref[b, pl.ds(r, 1), :], 0.0))
            nz = jnp.sum(jnp.where(lm, z_ref[b, pl.ds(r, 1), :], 0.0))
            write_kp(b, i, nx, ny, nz)
            out.extend([dists, nx, ny, nz])
        return tuple(out)

    lax.fori_loop(1, K, step, tuple(carry0))


def _fps(xyz):
    Xs = xyz[..., 0].reshape(B, _SIDE, _SIDE)
    Ys = xyz[..., 1].reshape(B, _SIDE, _SIDE)
    Zs = xyz[..., 2].reshape(B, _SIDE, _SIDE)
    return pl.pallas_call(
        _fps_body,
        out_shape=jax.ShapeDtypeStruct((B, K, _SIDE), jnp.float32),
    )(Xs, Ys, Zs)


def _prep_body(kp_ref, ptsT_ref, wc_ref, o0_ref, o1_ref, c0_ref, c1_ref, *,
               kb):
    kp = kp_ref[0]
    ptsT = ptsT_ref[0]
    kx = kp[:, 0]
    ky = kp[:, 1]
    kz = kp[:, 2]
    nk2 = (kx * kx + kz * kz) + ky * ky
    px = ptsT[0, :]
    py = ptsT[1, :]
    pz = ptsT[2, :]
    n2 = (px * px + pz * pz) + py * py
    dot = jnp.dot(kp, ptsT[0:3, :])
    d2 = (nk2[:, None] + n2[None, :]) - 2.0 * dot
    idx = lax.broadcasted_iota(jnp.int32, (kb, NPTS), 1)
    wc = wc_ref[...]
    for r, o_ref, c_ref in ((RADII[0], o0_ref, c0_ref),
                            (RADII[1], o1_ref, c1_ref)):
        mask = d2 < r * r
        order = jnp.where(mask, idx, jnp.int32(NPTS))
        o_ref[0] = order.reshape(kb, _NCHUNK, 128)
        cnt = jnp.dot(jnp.where(mask, 1.0, 0.0), wc)
        c_ref[0] = cnt.astype(jnp.int32)


def _prep(kp3, ptsT4, kb=64):
    r_iota = lax.broadcasted_iota(jnp.int32, (NPTS, _NCHUNK), 0)
    c_iota = lax.broadcasted_iota(jnp.int32, (NPTS, _NCHUNK), 1)
    wc = jnp.where(r_iota // _NCHUNK == c_iota, 1.0, 0.0)
    body = functools.partial(_prep_body, kb=kb)
    return pl.pallas_call(
        body,
        grid=(B, K // kb),
        in_specs=[
            pl.BlockSpec((1, kb, 3), lambda b, i: (b, i, 0)),
            pl.BlockSpec((1, 4, NPTS), lambda b, i: (b, 0, 0)),
            pl.BlockSpec((NPTS, _NCHUNK), lambda b, i: (0, 0)),
        ],
        out_specs=[
            pl.BlockSpec((1, kb, _NCHUNK, 128), lambda b, i: (b, i, 0, 0)),
            pl.BlockSpec((1, kb, _NCHUNK, 128), lambda b, i: (b, i, 0, 0)),
            pl.BlockSpec((1, kb, _NCHUNK), lambda b, i: (b, i, 0)),
            pl.BlockSpec((1, kb, _NCHUNK), lambda b, i: (b, i, 0)),
        ],
        out_shape=[
            jax.ShapeDtypeStruct((B, K, _NCHUNK, 128), jnp.int32),
            jax.ShapeDtypeStruct((B, K, _NCHUNK, 128), jnp.int32),
            jax.ShapeDtypeStruct((B, K, _NCHUNK), jnp.int32),
            jax.ShapeDtypeStruct((B, K, _NCHUNK), jnp.int32),
        ],
    )(kp3, ptsT4, wc)


def _extract_sc(order, counts, kp4, pts4, nsample):
    countsf = counts.reshape(B, K * _NCHUNK)
    kpf = kp4.reshape(B, K * 4)
    ptsf = pts4.reshape(B, NPTS * 4)
    mesh = plsc.VectorSubcoreMesh(core_axis_name="c", subcore_axis_name="s")
    nspad = nsample * 4

    def body(order_hbm, counts_hbm, kp_hbm, pts_hbm, g_hbm,
             pts_v, counts_v, kp_v, g_v, chunkdata,
             stage_a, stage_b, cbuf_a, cbuf_b, idxbuf, sem_a, sem_b):
        c = lax.axis_index("c")
        s = lax.axis_index("s")
        base = s * _ROWS
        pltpu.sync_copy(pts_hbm.at[c], pts_v)
        pltpu.sync_copy(counts_hbm.at[c, pl.ds(base * _NCHUNK,
                                               _ROWS * _NCHUNK)], counts_v)
        pltpu.sync_copy(kp_hbm.at[c, pl.ds(base * 4, _ROWS * 4)], kp_v)

        iota = lax.broadcasted_iota(jnp.int32, (16,), 0)
        nvec = jnp.full((16,), NPTS, dtype=jnp.int32)
        neg = jnp.full((16,), -(1 << 30), dtype=jnp.int32)

        def cextract(cbuf, j):
            cvec = cbuf[pl.ds(j, 16)]
            return jnp.max(jnp.where(iota == 0, cvec, neg))

        def build_and_fire(r, cbuf, stage, sem):
            cpos = jnp.int32(0)
            for v8 in range(8):
                cv = counts_v[pl.ds(r * _NCHUNK + v8 * 16, 16)]
                m = cv > 0
                mi = jnp.where(m, 1, 0)
                incl = plsc.cumsum(mi)
                plsc.store_scatter(cbuf, [cpos + (incl - mi)],
                                   iota + v8 * 16, mask=m)
                cpos = cpos + jnp.max(incl)
            ch0 = jnp.clip(cextract(cbuf, 0), 0, _NCHUNK - 1)
            ch1 = jnp.clip(cextract(cbuf, 1), 0, _NCHUNK - 1)
            pltpu.async_copy(order_hbm.at[c, base + r, ch0],
                             stage.at[pl.ds(0, 128)], sem)
            pltpu.async_copy(order_hbm.at[c, base + r, ch1],
                             stage.at[pl.ds(128, 128)], sem)
            return cpos

        def process(r, cbuf, stage, sem, nch):
            pltpu.make_async_copy(order_hbm.at[c, 0, 0],
                                  stage.at[pl.ds(0, 128)], sem).wait()
            pltpu.make_async_copy(order_hbm.at[c, 0, 0],
                                  stage.at[pl.ds(128, 128)], sem).wait()
            for t in range(nsample // 16 + 1):
                idxbuf[pl.ds(t * 16, 16)] = nvec
            pos = jnp.int32(0)
            for t in range(2):
                live = t < nch
                for v in range(8):
                    ov = stage[pl.ds(t * 128 + v * 16, 16)]
                    m = jnp.logical_and(ov < NPTS, live)
                    mi = jnp.where(m, 1, 0)
                    incl = plsc.cumsum(mi)
                    plsc.store_scatter(idxbuf, [pos + (incl - mi)], ov,
                                       mask=m)
                    pos = pos + jnp.max(incl)

            def chunk_cond(carry):
                j, pos = carry
                return jnp.logical_and(j < nch, pos < nsample)

            def chunk_body(carry):
                j, pos = carry
                chunk = cextract(cbuf, j)
                pltpu.sync_copy(order_hbm.at[c, base + r, chunk], chunkdata)
                for v in range(8):
                    ov = chunkdata[pl.ds(v * 16, 16)]
                    m = ov < NPTS
                    mi = jnp.where(m, 1, 0)
                    incl = plsc.cumsum(mi)
                    plsc.store_scatter(idxbuf, [pos + (incl - mi)], ov,
                                       mask=m)
                    pos = pos + jnp.max(incl)
                return (j + 1, pos)

            lax.while_loop(chunk_cond, chunk_body, (jnp.int32(2), pos))

            kpvec = plsc.load_gather(kp_v, [r * 4 + iota % 4])
            for sg in range(nsample // 4):
                i0 = plsc.load_gather(idxbuf, [iota // 4 + sg * 4])
                msk = i0 < NPTS
                i0c = jnp.where(msk, i0, 0)
                val = plsc.load_gather(pts_v, [i0c * 4 + iota % 4])
                g = jnp.where(msk, val - kpvec, 0.0)
                g_v[pl.ds(r * nspad + sg * 16, 16)] = g

        nch0 = build_and_fire(0, cbuf_a, stage_a, sem_a)

        def pair_body(gidx, nch_a):
            r0 = gidx * 2
            nch_b = build_and_fire(r0 + 1, cbuf_b, stage_b, sem_b)
            process(r0, cbuf_a, stage_a, sem_a, nch_a)
            r2 = jnp.minimum(r0 + 2, _ROWS - 1)
            nch_a2 = build_and_fire(r2, cbuf_a, stage_a, sem_a)
            process(r0 + 1, cbuf_b, stage_b, sem_b, nch_b)
            return nch_a2

        lax.fori_loop(0, _ROWS // 2, pair_body, nch0)
        pltpu.make_async_copy(order_hbm.at[c, 0, 0],
                              stage_a.at[pl.ds(0, 128)], sem_a).wait()
        pltpu.make_async_copy(order_hbm.at[c, 0, 0],
                              stage_a.at[pl.ds(128, 128)], sem_a).wait()
        pltpu.sync_copy(g_v, g_hbm.at[c, pl.ds(base * nspad, _ROWS * nspad)])

    run = pl.kernel(
        body,
        out_type=jax.ShapeDtypeStruct((B, K * nspad), jnp.float32),
        mesh=mesh,
        compiler_params=pltpu.CompilerParams(needs_layout_passes=False),
        scratch_types=[
            pltpu.VMEM((NPTS * 4,), jnp.float32),
            pltpu.VMEM((_ROWS * _NCHUNK,), jnp.int32),
            pltpu.VMEM((_ROWS * 4,), jnp.float32),
            pltpu.VMEM((_ROWS * nspad,), jnp.float32),
            pltpu.VMEM((_NCHUNK,), jnp.int32),
            pltpu.VMEM((256,), jnp.int32),
            pltpu.VMEM((256,), jnp.int32),
            pltpu.VMEM((160,), jnp.int32),
            pltpu.VMEM((160,), jnp.int32),
            pltpu.VMEM((_SP,), jnp.int32),
            pltpu.SemaphoreType.DMA,
            pltpu.SemaphoreType.DMA,
        ],
    )
    return run(order, countsf, kpf, ptsf).reshape(B, K, nspad)


def _mlp_body(g_ref, wa_ref, wb_ref, out_ref, *, nsample, kb):
    g_all = g_ref[0]
    wa = wa_ref[...]
    wb = wb_ref[...]
    pooled = jnp.zeros((kb, 16), dtype=jnp.float32)
    for n in range(nsample):
        g = g_all[:, n * 4:(n + 1) * 4]
        h1 = jnp.maximum(jnp.dot(g, wa), 0.0)
        h2 = jnp.maximum(jnp.dot(h1, wb), 0.0)
        pooled = jnp.maximum(pooled, h2)
    out_ref[0] = pooled


def _mlp(g, Wa, Wb, nsample, kb=512):
    body = functools.partial(_mlp_body, nsample=nsample, kb=kb)
    return pl.pallas_call(
        body,
        grid=(B, K // kb),
        in_specs=[
            pl.BlockSpec((1, kb, nsample * 4), lambda b, i: (b, i, 0)),
            pl.BlockSpec((4, 16), lambda b, i: (0, 0)),
            pl.BlockSpec((16, 16), lambda b, i: (0, 0)),
        ],
        out_specs=pl.BlockSpec((1, kb, 16), lambda b, i: (b, i, 0)),
        out_shape=jax.ShapeDtypeStruct((B, K, 16), jnp.float32),
    )(g, Wa, Wb)


def kernel(points, W0a, W0b, W1a, W1b):
    pts = points.reshape(B, NPTS, 5)
    xyz = pts[:, :, 1:4]
    kp_pad = _fps(xyz)
    kp3 = kp_pad[:, :, :3]
    ptsT4 = pts[:, :, 1:5].transpose(0, 2, 1)
    o0, o1, c0, c1 = _prep(kp3, ptsT4)
    kp4 = kp_pad[:, :, :4] * jnp.array([1.0, 1.0, 1.0, 0.0])
    pts4 = pts[:, :, 1:5]
    g0 = _extract_sc(o0, c0, kp4, pts4, NSAMPLE[0])
    g1 = _extract_sc(o1, c1, kp4, pts4, NSAMPLE[1])
    f0 = _mlp(g0, W0a, W0b, NSAMPLE[0])
    f1 = _mlp(g1, W1a, W1b, NSAMPLE[1])
    point_features = jnp.concatenate([f0, f1], axis=2).reshape(B * K, 32)
    bcol = jnp.repeat(jnp.arange(B, dtype=jnp.float32), K)[:, None]
    point_coords = jnp.concatenate([bcol, kp3.reshape(B * K, 3)], axis=1)
    return point_features, point_coords


# speedup vs baseline: 1.2220x; 1.0035x over previous
import functools

import jax
import jax.numpy as jnp
from jax import lax
from jax.experimental import pallas as pl
from jax.experimental.pallas import tpu as pltpu
from jax.experimental.pallas import tpu_sc as plsc

B = 2
NPTS = 16384
K = 2048
RADII = (0.4, 0.8)
NSAMPLE = (16, 32)
_SIDE = 128
_NCHUNK = 128
_ROWS = 128
_SP = 304


def _fps_body(x_ref, y_ref, z_ref, kp_ref):
    rows = lax.broadcasted_iota(jnp.int32, (_SIDE, _SIDE), 0)
    cols = lax.broadcasted_iota(jnp.int32, (_SIDE, _SIDE), 1)
    flat = rows * _SIDE + cols
    lane = lax.broadcasted_iota(jnp.int32, (1, _SIDE), 1)
    XYZ = [(x_ref[b], y_ref[b], z_ref[b]) for b in range(B)]

    def write_kp(b, i, xs, ys, zs):
        row = jnp.where(lane == 0, xs,
                        jnp.where(lane == 1, ys,
                                  jnp.where(lane == 2, zs, 0.0)))
        kp_ref[b, pl.ds(i, 1), :] = row

    carry0 = []
    for b in range(B):
        X, Y, Z = XYZ[b]
        xs0 = X[0, 0]
        ys0 = Y[0, 0]
        zs0 = Z[0, 0]
        write_kp(b, 0, xs0, ys0, zs0)
        dists0 = jnp.full((_SIDE, _SIDE), 1e10, dtype=jnp.float32)
        carry0.extend([dists0, xs0, ys0, zs0])

    def step(i, carry):
        out = []
        for b in range(B):
            dists, xs, ys, zs = carry[4 * b:4 * b + 4]
            X, Y, Z = XYZ[b]
            dx = X - xs
            dy = Y - ys
            dz = Z - zs
            d = (dx * dx + dz * dz) + dy * dy
            dists = jnp.minimum(dists, d)
            m = jnp.max(dists)
            nxt = jnp.min(jnp.where(dists == m, flat, jnp.int32(1 << 30)))
            r = nxt // _SIDE
            col = nxt % _SIDE
            lm = lane == col
            nx = jnp.sum(jnp.where(lm, x_ref[b, pl.ds(r, 1), :], 0.0))
            ny = jnp.sum(jnp.where(lm, y_ref[b, pl.ds(r, 1), :], 0.0))
            nz = jnp.sum(jnp.where(lm, z_ref[b, pl.ds(r, 1), :], 0.0))
            write_kp(b, i, nx, ny, nz)
            out.extend([dists, nx, ny, nz])
        return tuple(out)

    lax.fori_loop(1, K, step, tuple(carry0))


def _fps(xyz):
    Xs = xyz[..., 0].reshape(B, _SIDE, _SIDE)
    Ys = xyz[..., 1].reshape(B, _SIDE, _SIDE)
    Zs = xyz[..., 2].reshape(B, _SIDE, _SIDE)
    return pl.pallas_call(
        _fps_body,
        out_shape=jax.ShapeDtypeStruct((B, K, _SIDE), jnp.float32),
    )(Xs, Ys, Zs)


def _prep_body(kp_ref, ptsT_ref, wc_ref, o0_ref, o1_ref, c0_ref, c1_ref, *,
               kb):
    kp = kp_ref[0]
    ptsT = ptsT_ref[0]
    kx = kp[:, 0]
    ky = kp[:, 1]
    kz = kp[:, 2]
    nk2 = (kx * kx + kz * kz) + ky * ky
    px = ptsT[0, :]
    py = ptsT[1, :]
    pz = ptsT[2, :]
    n2 = (px * px + pz * pz) + py * py
    dot = jnp.dot(kp, ptsT[0:3, :])
    d2 = (nk2[:, None] + n2[None, :]) - 2.0 * dot
    idx = lax.broadcasted_iota(jnp.int32, (kb, NPTS), 1)
    wc = wc_ref[...]
    for r, o_ref, c_ref in ((RADII[0], o0_ref, c0_ref),
                            (RADII[1], o1_ref, c1_ref)):
        mask = d2 < r * r
        order = jnp.where(mask, idx, jnp.int32(NPTS))
        o_ref[0] = order.reshape(kb, _NCHUNK, 128)
        cnt = jnp.dot(jnp.where(mask, 1.0, 0.0), wc)
        c_ref[0] = cnt.astype(jnp.int32)


def _prep(kp3, ptsT4, kb=128):
    r_iota = lax.broadcasted_iota(jnp.int32, (NPTS, _NCHUNK), 0)
    c_iota = lax.broadcasted_iota(jnp.int32, (NPTS, _NCHUNK), 1)
    wc = jnp.where(r_iota // _NCHUNK == c_iota, 1.0, 0.0)
    body = functools.partial(_prep_body, kb=kb)
    return pl.pallas_call(
        body,
        grid=(B, K // kb),
        in_specs=[
            pl.BlockSpec((1, kb, 3), lambda b, i: (b, i, 0)),
            pl.BlockSpec((1, 4, NPTS), lambda b, i: (b, 0, 0)),
            pl.BlockSpec((NPTS, _NCHUNK), lambda b, i: (0, 0)),
        ],
        out_specs=[
            pl.BlockSpec((1, kb, _NCHUNK, 128), lambda b, i: (b, i, 0, 0)),
            pl.BlockSpec((1, kb, _NCHUNK, 128), lambda b, i: (b, i, 0, 0)),
            pl.BlockSpec((1, kb, _NCHUNK), lambda b, i: (b, i, 0)),
            pl.BlockSpec((1, kb, _NCHUNK), lambda b, i: (b, i, 0)),
        ],
        out_shape=[
            jax.ShapeDtypeStruct((B, K, _NCHUNK, 128), jnp.int32),
            jax.ShapeDtypeStruct((B, K, _NCHUNK, 128), jnp.int32),
            jax.ShapeDtypeStruct((B, K, _NCHUNK), jnp.int32),
            jax.ShapeDtypeStruct((B, K, _NCHUNK), jnp.int32),
        ],
    )(kp3, ptsT4, wc)


def _extract_sc(order, counts, kp4, pts4, nsample):
    countsf = counts.reshape(B, K * _NCHUNK)
    kpf = kp4.reshape(B, K * 4)
    ptsf = pts4.reshape(B, NPTS * 4)
    mesh = plsc.VectorSubcoreMesh(core_axis_name="c", subcore_axis_name="s")
    nspad = nsample * 4

    def body(order_hbm, counts_hbm, kp_hbm, pts_hbm, g_hbm,
             pts_v, counts_v, kp_v, g_v, chunkdata,
             stage_a, stage_b, cbuf_a, cbuf_b, idxbuf, sem_a, sem_b):
        c = lax.axis_index("c")
        s = lax.axis_index("s")
        base = s * _ROWS
        pltpu.sync_copy(pts_hbm.at[c], pts_v)
        pltpu.sync_copy(counts_hbm.at[c, pl.ds(base * _NCHUNK,
                                               _ROWS * _NCHUNK)], counts_v)
        pltpu.sync_copy(kp_hbm.at[c, pl.ds(base * 4, _ROWS * 4)], kp_v)

        iota = lax.broadcasted_iota(jnp.int32, (16,), 0)
        nvec = jnp.full((16,), NPTS, dtype=jnp.int32)
        neg = jnp.full((16,), -(1 << 30), dtype=jnp.int32)

        def cextract(cbuf, j):
            cvec = cbuf[pl.ds(j, 16)]
            return jnp.max(jnp.where(iota == 0, cvec, neg))

        def build_and_fire(r, cbuf, stage, sem):
            cpos = jnp.int32(0)
            for v8 in range(8):
                cv = counts_v[pl.ds(r * _NCHUNK + v8 * 16, 16)]
                m = cv > 0
                mi = jnp.where(m, 1, 0)
                incl = plsc.cumsum(mi)
                plsc.store_scatter(cbuf, [cpos + (incl - mi)],
                                   iota + v8 * 16, mask=m)
                cpos = cpos + jnp.max(incl)
            ch0 = jnp.clip(cextract(cbuf, 0), 0, _NCHUNK - 1)
            ch1 = jnp.clip(cextract(cbuf, 1), 0, _NCHUNK - 1)
            pltpu.async_copy(order_hbm.at[c, base + r, ch0],
                             stage.at[pl.ds(0, 128)], sem)
            pltpu.async_copy(order_hbm.at[c, base + r, ch1],
                             stage.at[pl.ds(128, 128)], sem)
            return cpos

        def process(r, cbuf, stage, sem, nch):
            pltpu.make_async_copy(order_hbm.at[c, 0, 0],
                                  stage.at[pl.ds(0, 128)], sem).wait()
            pltpu.make_async_copy(order_hbm.at[c, 0, 0],
                                  stage.at[pl.ds(128, 128)], sem).wait()
            for t in range(nsample // 16 + 1):
                idxbuf[pl.ds(t * 16, 16)] = nvec
            pos = jnp.int32(0)
            for t in range(2):
                live = t < nch
                for v in range(8):
                    ov = stage[pl.ds(t * 128 + v * 16, 16)]
                    m = jnp.logical_and(ov < NPTS, live)
                    mi = jnp.where(m, 1, 0)
                    incl = plsc.cumsum(mi)
                    plsc.store_scatter(idxbuf, [pos + (incl - mi)], ov,
                                       mask=m)
                    pos = pos + jnp.max(incl)

            def chunk_cond(carry):
                j, pos = carry
                return jnp.logical_and(j < nch, pos < nsample)

            def chunk_body(carry):
                j, pos = carry
                chunk = cextract(cbuf, j)
                pltpu.sync_copy(order_hbm.at[c, base + r, chunk], chunkdata)
                for v in range(8):
                    ov = chunkdata[pl.ds(v * 16, 16)]
                    m = ov < NPTS
                    mi = jnp.where(m, 1, 0)
                    incl = plsc.cumsum(mi)
                    plsc.store_scatter(idxbuf, [pos + (incl - mi)], ov,
                                       mask=m)
                    pos = pos + jnp.max(incl)
                return (j + 1, pos)

            lax.while_loop(chunk_cond, chunk_body, (jnp.int32(2), pos))

            kpvec = plsc.load_gather(kp_v, [r * 4 + iota % 4])
            for sg in range(nsample // 4):
                i0 = plsc.load_gather(idxbuf, [iota // 4 + sg * 4])
                msk = i0 < NPTS
                i0c = jnp.where(msk, i0, 0)
                val = plsc.load_gather(pts_v, [i0c * 4 + iota % 4])
                g = jnp.where(msk, val - kpvec, 0.0)
                g_v[pl.ds(r * nspad + sg * 16, 16)] = g

        nch0 = build_and_fire(0, cbuf_a, stage_a, sem_a)

        def pair_body(gidx, nch_a):
            r0 = gidx * 2
            nch_b = build_and_fire(r0 + 1, cbuf_b, stage_b, sem_b)
            process(r0, cbuf_a, stage_a, sem_a, nch_a)
            r2 = jnp.minimum(r0 + 2, _ROWS - 1)
            nch_a2 = build_and_fire(r2, cbuf_a, stage_a, sem_a)
            process(r0 + 1, cbuf_b, stage_b, sem_b, nch_b)
            return nch_a2

        lax.fori_loop(0, _ROWS // 2, pair_body, nch0)
        pltpu.make_async_copy(order_hbm.at[c, 0, 0],
                              stage_a.at[pl.ds(0, 128)], sem_a).wait()
        pltpu.make_async_copy(order_hbm.at[c, 0, 0],
                              stage_a.at[pl.ds(128, 128)], sem_a).wait()
        pltpu.sync_copy(g_v, g_hbm.at[c, pl.ds(base * nspad, _ROWS * nspad)])

    run = pl.kernel(
        body,
        out_type=jax.ShapeDtypeStruct((B, K * nspad), jnp.float32),
        mesh=mesh,
        compiler_params=pltpu.CompilerParams(needs_layout_passes=False),
        scratch_types=[
            pltpu.VMEM((NPTS * 4,), jnp.float32),
            pltpu.VMEM((_ROWS * _NCHUNK,), jnp.int32),
            pltpu.VMEM((_ROWS * 4,), jnp.float32),
            pltpu.VMEM((_ROWS * nspad,), jnp.float32),
            pltpu.VMEM((_NCHUNK,), jnp.int32),
            pltpu.VMEM((256,), jnp.int32),
            pltpu.VMEM((256,), jnp.int32),
            pltpu.VMEM((160,), jnp.int32),
            pltpu.VMEM((160,), jnp.int32),
            pltpu.VMEM((_SP,), jnp.int32),
            pltpu.SemaphoreType.DMA,
            pltpu.SemaphoreType.DMA,
        ],
    )
    return run(order, countsf, kpf, ptsf).reshape(B, K, nspad)


def _mlp_body(g_ref, wa_ref, wb_ref, out_ref, *, nsample, kb):
    g_all = g_ref[0]
    wa = wa_ref[...]
    wb = wb_ref[...]
    pooled = jnp.zeros((kb, 16), dtype=jnp.float32)
    for n in range(nsample):
        g = g_all[:, n * 4:(n + 1) * 4]
        h1 = jnp.maximum(jnp.dot(g, wa), 0.0)
        h2 = jnp.maximum(jnp.dot(h1, wb), 0.0)
        pooled = jnp.maximum(pooled, h2)
    out_ref[0] = pooled


def _mlp(g, Wa, Wb, nsample, kb=512):
    body = functools.partial(_mlp_body, nsample=nsample, kb=kb)
    return pl.pallas_call(
        body,
        grid=(B, K // kb),
        in_specs=[
            pl.BlockSpec((1, kb, nsample * 4), lambda b, i: (b, i, 0)),
            pl.BlockSpec((4, 16), lambda b, i: (0, 0)),
            pl.BlockSpec((16, 16), lambda b, i: (0, 0)),
        ],
        out_specs=pl.BlockSpec((1, kb, 16), lambda b, i: (b, i, 0)),
        out_shape=jax.ShapeDtypeStruct((B, K, 16), jnp.float32),
    )(g, Wa, Wb)


def kernel(points, W0a, W0b, W1a, W1b):
    pts = points.reshape(B, NPTS, 5)
    xyz = pts[:, :, 1:4]
    kp_pad = _fps(xyz)
    kp3 = kp_pad[:, :, :3]
    ptsT4 = pts[:, :, 1:5].transpose(0, 2, 1)
    o0, o1, c0, c1 = _prep(kp3, ptsT4)
    kp4 = kp_pad[:, :, :4] * jnp.array([1.0, 1.0, 1.0, 0.0])
    pts4 = pts[:, :, 1:5]
    g0 = _extract_sc(o0, c0, kp4, pts4, NSAMPLE[0])
    g1 = _extract_sc(o1, c1, kp4, pts4, NSAMPLE[1])
    f0 = _mlp(g0, W0a, W0b, NSAMPLE[0])
    f1 = _mlp(g1, W1a, W1b, NSAMPLE[1])
    point_features = jnp.concatenate([f0, f1], axis=2).reshape(B * K, 32)
    bcol = jnp.repeat(jnp.arange(B, dtype=jnp.float32), K)[:, None]
    point_coords = jnp.concatenate([bcol, kp3.reshape(B * K, 3)], axis=1)
    return point_features, point_coords


# speedup vs baseline: 1.2231x; 1.0009x over previous
import functools

import jax
import jax.numpy as jnp
from jax import lax
from jax.experimental import pallas as pl
from jax.experimental.pallas import tpu as pltpu
from jax.experimental.pallas import tpu_sc as plsc

B = 2
NPTS = 16384
K = 2048
RADII = (0.4, 0.8)
NSAMPLE = (16, 32)
_SIDE = 128
_NCHUNK = 128
_ROWS = 128
_SP = 304


def _fps_body(x_ref, y_ref, z_ref, kp_ref):
    rows = lax.broadcasted_iota(jnp.int32, (_SIDE, _SIDE), 0)
    cols = lax.broadcasted_iota(jnp.int32, (_SIDE, _SIDE), 1)
    flat = rows * _SIDE + cols
    lane = lax.broadcasted_iota(jnp.int32, (1, _SIDE), 1)
    XYZ = [(x_ref[b], y_ref[b], z_ref[b]) for b in range(B)]

    def write_kp(b, i, xs, ys, zs):
        row = jnp.where(lane == 0, xs,
                        jnp.where(lane == 1, ys,
                                  jnp.where(lane == 2, zs, 0.0)))
        kp_ref[b, pl.ds(i, 1), :] = row

    carry0 = []
    for b in range(B):
        X, Y, Z = XYZ[b]
        xs0 = X[0, 0]
        ys0 = Y[0, 0]
        zs0 = Z[0, 0]
        write_kp(b, 0, xs0, ys0, zs0)
        dists0 = jnp.full((_SIDE, _SIDE), 1e10, dtype=jnp.float32)
        carry0.extend([dists0, xs0, ys0, zs0])

    def step(i, carry):
        out = []
        for b in range(B):
            dists, xs, ys, zs = carry[4 * b:4 * b + 4]
            X, Y, Z = XYZ[b]
            dx = X - xs
            dy = Y - ys
            dz = Z - zs
            d = (dx * dx + dz * dz) + dy * dy
            dists = jnp.minimum(dists, d)
            m = jnp.max(dists)
            nxt = jnp.min(jnp.where(dists == m, flat, jnp.int32(1 << 30)))
            r = nxt // _SIDE
            col = nxt % _SIDE
            lm = lane == col
            nx = jnp.sum(jnp.where(lm, x_ref[b, pl.ds(r, 1), :], 0.0))
            ny = jnp.sum(jnp.where(lm, y_ref[b, pl.ds(r, 1), :], 0.0))
            nz = jnp.sum(jnp.where(lm, z_ref[b, pl.ds(r, 1), :], 0.0))
            write_kp(b, i, nx, ny, nz)
            out.extend([dists, nx, ny, nz])
        return tuple(out)

    lax.fori_loop(1, K, step, tuple(carry0))


def _fps(xyz):
    Xs = xyz[..., 0].reshape(B, _SIDE, _SIDE)
    Ys = xyz[..., 1].reshape(B, _SIDE, _SIDE)
    Zs = xyz[..., 2].reshape(B, _SIDE, _SIDE)
    return pl.pallas_call(
        _fps_body,
        out_shape=jax.ShapeDtypeStruct((B, K, _SIDE), jnp.float32),
    )(Xs, Ys, Zs)


def _prep_body(kp_ref, ptsT_ref, wc_ref, o0_ref, o1_ref, c0_ref, c1_ref, *,
               kb):
    kp = kp_ref[0]
    ptsT = ptsT_ref[0]
    kx = kp[:, 0]
    ky = kp[:, 1]
    kz = kp[:, 2]
    nk2 = (kx * kx + kz * kz) + ky * ky
    px = ptsT[0, :]
    py = ptsT[1, :]
    pz = ptsT[2, :]
    n2 = (px * px + pz * pz) + py * py
    dot = jnp.dot(kp, ptsT[0:3, :])
    d2 = (nk2[:, None] + n2[None, :]) - 2.0 * dot
    idx = lax.broadcasted_iota(jnp.int32, (kb, NPTS), 1)
    wc = wc_ref[...]
    for r, o_ref, c_ref in ((RADII[0], o0_ref, c0_ref),
                            (RADII[1], o1_ref, c1_ref)):
        mask = d2 < r * r
        order = jnp.where(mask, idx, jnp.int32(NPTS))
        o_ref[0] = order.reshape(kb, _NCHUNK, 128)
        cnt = jnp.dot(jnp.where(mask, 1.0, 0.0), wc)
        c_ref[0] = cnt.astype(jnp.int32)


def _prep(kp3, ptsT4, kb=128):
    r_iota = lax.broadcasted_iota(jnp.int32, (NPTS, _NCHUNK), 0)
    c_iota = lax.broadcasted_iota(jnp.int32, (NPTS, _NCHUNK), 1)
    wc = jnp.where(r_iota // _NCHUNK == c_iota, 1.0, 0.0)
    body = functools.partial(_prep_body, kb=kb)
    return pl.pallas_call(
        body,
        grid=(B, K // kb),
        in_specs=[
            pl.BlockSpec((1, kb, 3), lambda b, i: (b, i, 0)),
            pl.BlockSpec((1, 4, NPTS), lambda b, i: (b, 0, 0)),
            pl.BlockSpec((NPTS, _NCHUNK), lambda b, i: (0, 0)),
        ],
        out_specs=[
            pl.BlockSpec((1, kb, _NCHUNK, 128), lambda b, i: (b, i, 0, 0)),
            pl.BlockSpec((1, kb, _NCHUNK, 128), lambda b, i: (b, i, 0, 0)),
            pl.BlockSpec((1, kb, _NCHUNK), lambda b, i: (b, i, 0)),
            pl.BlockSpec((1, kb, _NCHUNK), lambda b, i: (b, i, 0)),
        ],
        out_shape=[
            jax.ShapeDtypeStruct((B, K, _NCHUNK, 128), jnp.int32),
            jax.ShapeDtypeStruct((B, K, _NCHUNK, 128), jnp.int32),
            jax.ShapeDtypeStruct((B, K, _NCHUNK), jnp.int32),
            jax.ShapeDtypeStruct((B, K, _NCHUNK), jnp.int32),
        ],
    )(kp3, ptsT4, wc)


def _extract_sc(order, counts, kp4, pts4, nsample):
    countsf = counts.reshape(B, K * _NCHUNK)
    kpf = kp4.reshape(B, K * 4)
    ptsf = pts4.reshape(B, NPTS * 4)
    mesh = plsc.VectorSubcoreMesh(core_axis_name="c", subcore_axis_name="s")
    nspad = nsample * 4

    def body(order_hbm, counts_hbm, kp_hbm, pts_hbm, g_hbm,
             pts_v, counts_v, kp_v, g_v, chunkdata,
             stage_a, stage_b, cbuf_a, cbuf_b, idxbuf, sem_a, sem_b):
        c = lax.axis_index("c")
        s = lax.axis_index("s")
        base = s * _ROWS
        pltpu.sync_copy(pts_hbm.at[c], pts_v)
        pltpu.sync_copy(counts_hbm.at[c, pl.ds(base * _NCHUNK,
                                               _ROWS * _NCHUNK)], counts_v)
        pltpu.sync_copy(kp_hbm.at[c, pl.ds(base * 4, _ROWS * 4)], kp_v)

        iota = lax.broadcasted_iota(jnp.int32, (16,), 0)
        nvec = jnp.full((16,), NPTS, dtype=jnp.int32)
        neg = jnp.full((16,), -(1 << 30), dtype=jnp.int32)

        def cextract(cbuf, j):
            cvec = cbuf[pl.ds(j, 16)]
            return jnp.max(jnp.where(iota == 0, cvec, neg))

        def build_and_fire(r, cbuf, stage, sem):
            cpos = jnp.int32(0)
            for v8 in range(8):
                cv = counts_v[pl.ds(r * _NCHUNK + v8 * 16, 16)]
                m = cv > 0
                mi = jnp.where(m, 1, 0)
                incl = plsc.cumsum(mi)
                plsc.store_scatter(cbuf, [cpos + (incl - mi)],
                                   iota + v8 * 16, mask=m)
                cpos = cpos + jnp.max(incl)
            ch0 = jnp.clip(cextract(cbuf, 0), 0, _NCHUNK - 1)
            ch1 = jnp.clip(cextract(cbuf, 1), 0, _NCHUNK - 1)
            pltpu.async_copy(order_hbm.at[c, base + r, ch0],
                             stage.at[pl.ds(0, 128)], sem)
            pltpu.async_copy(order_hbm.at[c, base + r, ch1],
                             stage.at[pl.ds(128, 128)], sem)
            return cpos

        def process(r, cbuf, stage, sem, nch):
            pltpu.make_async_copy(order_hbm.at[c, 0, 0],
                                  stage.at[pl.ds(0, 128)], sem).wait()
            pltpu.make_async_copy(order_hbm.at[c, 0, 0],
                                  stage.at[pl.ds(128, 128)], sem).wait()
            def full_path():
                for t in range(nsample // 16 + 1):
                    idxbuf[pl.ds(t * 16, 16)] = nvec
                pos = jnp.int32(0)
                for t in range(2):
                    live = t < nch
                    for v in range(8):
                        ov = stage[pl.ds(t * 128 + v * 16, 16)]
                        m = jnp.logical_and(ov < NPTS, live)
                        mi = jnp.where(m, 1, 0)
                        incl = plsc.cumsum(mi)
                        plsc.store_scatter(idxbuf, [pos + (incl - mi)], ov,
                                           mask=m)
                        pos = pos + jnp.max(incl)

                def chunk_cond(carry):
                    j, pos = carry
                    return jnp.logical_and(j < nch, pos < nsample)

                def chunk_body(carry):
                    j, pos = carry
                    chunk = cextract(cbuf, j)
                    pltpu.sync_copy(order_hbm.at[c, base + r, chunk],
                                    chunkdata)
                    for v in range(8):
                        ov = chunkdata[pl.ds(v * 16, 16)]
                        m = ov < NPTS
                        mi = jnp.where(m, 1, 0)
                        incl = plsc.cumsum(mi)
                        plsc.store_scatter(idxbuf, [pos + (incl - mi)], ov,
                                           mask=m)
                        pos = pos + jnp.max(incl)
                    return (j + 1, pos)

                lax.while_loop(chunk_cond, chunk_body, (jnp.int32(2), pos))

                kpvec = plsc.load_gather(kp_v, [r * 4 + iota % 4])
                for sg in range(nsample // 4):
                    i0 = plsc.load_gather(idxbuf, [iota // 4 + sg * 4])
                    msk = i0 < NPTS
                    i0c = jnp.where(msk, i0, 0)
                    val = plsc.load_gather(pts_v, [i0c * 4 + iota % 4])
                    g = jnp.where(msk, val - kpvec, 0.0)
                    g_v[pl.ds(r * nspad + sg * 16, 16)] = g
                return 0

            def empty_path():
                zero = jnp.zeros((16,), dtype=jnp.float32)
                for sg in range(nsample // 4):
                    g_v[pl.ds(r * nspad + sg * 16, 16)] = zero
                return 0

            lax.cond(nch > 0, full_path, empty_path)

        nch0 = build_and_fire(0, cbuf_a, stage_a, sem_a)

        def pair_body(gidx, nch_a):
            r0 = gidx * 2
            nch_b = build_and_fire(r0 + 1, cbuf_b, stage_b, sem_b)
            process(r0, cbuf_a, stage_a, sem_a, nch_a)
            r2 = jnp.minimum(r0 + 2, _ROWS - 1)
            nch_a2 = build_and_fire(r2, cbuf_a, stage_a, sem_a)
            process(r0 + 1, cbuf_b, stage_b, sem_b, nch_b)
            return nch_a2

        lax.fori_loop(0, _ROWS // 2, pair_body, nch0)
        pltpu.make_async_copy(order_hbm.at[c, 0, 0],
                              stage_a.at[pl.ds(0, 128)], sem_a).wait()
        pltpu.make_async_copy(order_hbm.at[c, 0, 0],
                              stage_a.at[pl.ds(128, 128)], sem_a).wait()
        pltpu.sync_copy(g_v, g_hbm.at[c, pl.ds(base * nspad, _ROWS * nspad)])

    run = pl.kernel(
        body,
        out_type=jax.ShapeDtypeStruct((B, K * nspad), jnp.float32),
        mesh=mesh,
        compiler_params=pltpu.CompilerParams(needs_layout_passes=False),
        scratch_types=[
            pltpu.VMEM((NPTS * 4,), jnp.float32),
            pltpu.VMEM((_ROWS * _NCHUNK,), jnp.int32),
            pltpu.VMEM((_ROWS * 4,), jnp.float32),
            pltpu.VMEM((_ROWS * nspad,), jnp.float32),
            pltpu.VMEM((_NCHUNK,), jnp.int32),
            pltpu.VMEM((256,), jnp.int32),
            pltpu.VMEM((256,), jnp.int32),
            pltpu.VMEM((160,), jnp.int32),
            pltpu.VMEM((160,), jnp.int32),
            pltpu.VMEM((_SP,), jnp.int32),
            pltpu.SemaphoreType.DMA,
            pltpu.SemaphoreType.DMA,
        ],
    )
    return run(order, countsf, kpf, ptsf).reshape(B, K, nspad)


def _mlp_body(g_ref, wa_ref, wb_ref, out_ref, *, nsample, kb):
    g_all = g_ref[0]
    wa = wa_ref[...]
    wb = wb_ref[...]
    pooled = jnp.zeros((kb, 16), dtype=jnp.float32)
    for n in range(nsample):
        g = g_all[:, n * 4:(n + 1) * 4]
        h1 = jnp.maximum(jnp.dot(g, wa), 0.0)
        h2 = jnp.maximum(jnp.dot(h1, wb), 0.0)
        pooled = jnp.maximum(pooled, h2)
    out_ref[0] = pooled


def _mlp(g, Wa, Wb, nsample, kb=512):
    body = functools.partial(_mlp_body, nsample=nsample, kb=kb)
    return pl.pallas_call(
        body,
        grid=(B, K // kb),
        in_specs=[
            pl.BlockSpec((1, kb, nsample * 4), lambda b, i: (b, i, 0)),
            pl.BlockSpec((4, 16), lambda b, i: (0, 0)),
            pl.BlockSpec((16, 16), lambda b, i: (0, 0)),
        ],
        out_specs=pl.BlockSpec((1, kb, 16), lambda b, i: (b, i, 0)),
        out_shape=jax.ShapeDtypeStruct((B, K, 16), jnp.float32),
    )(g, Wa, Wb)


def kernel(points, W0a, W0b, W1a, W1b):
    pts = points.reshape(B, NPTS, 5)
    xyz = pts[:, :, 1:4]
    kp_pad = _fps(xyz)
    kp3 = kp_pad[:, :, :3]
    ptsT4 = pts[:, :, 1:5].transpose(0, 2, 1)
    o0, o1, c0, c1 = _prep(kp3, ptsT4)
    kp4 = kp_pad[:, :, :4] * jnp.array([1.0, 1.0, 1.0, 0.0])
    pts4 = pts[:, :, 1:5]
    g0 = _extract_sc(o0, c0, kp4, pts4, NSAMPLE[0])
    g1 = _extract_sc(o1, c1, kp4, pts4, NSAMPLE[1])
    f0 = _mlp(g0, W0a, W0b, NSAMPLE[0])
    f1 = _mlp(g1, W1a, W1b, NSAMPLE[1])
    point_features = jnp.concatenate([f0, f1], axis=2).reshape(B * K, 32)
    bcol = jnp.repeat(jnp.arange(B, dtype=jnp.float32), K)[:, None]
    point_coords = jnp.concatenate([bcol, kp3.reshape(B * K, 3)], axis=1)
    return point_features, point_coords
